# baseline (speedup 1.0000x reference)
import functools

import jax
import jax.numpy as jnp
from jax import lax
from jax.experimental import pallas as pl
from jax.experimental.pallas import tpu as pltpu
from jax.experimental.pallas import tpu_sc as plsc

_NUM_CORES = 2
_CHUNK_ROWS = 256
_NBUF = 3


def _make_sc_copy(seq_len, n_embd, dtype):
    rows_per_core = seq_len // _NUM_CORES
    n_chunks = rows_per_core // _CHUNK_ROWS
    mesh = plsc.VectorSubcoreMesh(
        core_axis_name="c", subcore_axis_name="s"
    )

    @functools.partial(
        pl.kernel,
        mesh=mesh,
        out_type=jax.ShapeDtypeStruct((seq_len, n_embd), dtype),
        scratch_types=(
            [pltpu.VMEM_SHARED((_NBUF, _CHUNK_ROWS, n_embd), dtype)]
            + [pltpu.SemaphoreType.DMA] * (2 * _NBUF)
        ),
    )
    def sc_copy(table_hbm, out_hbm, spbuf, *sems):
        rsems = sems[:_NBUF]
        wsems = sems[_NBUF:]
        s = lax.axis_index("s")
        base = lax.axis_index("c") * rows_per_core

        def read(i):
            return pltpu.make_async_copy(
                table_hbm.at[pl.ds(base + i * _CHUNK_ROWS, _CHUNK_ROWS)],
                spbuf.at[i % _NBUF],
                rsems[i % _NBUF],
            )

        def write(i):
            return pltpu.make_async_copy(
                spbuf.at[i % _NBUF],
                out_hbm.at[pl.ds(base + i * _CHUNK_ROWS, _CHUNK_ROWS)],
                wsems[i % _NBUF],
            )

        @pl.when(s == 0)
        def _():
            for j in range(min(_NBUF - 1, n_chunks)):
                read(j).start()
            for i in range(n_chunks):
                read(i).wait()
                write(i).start()
                j = i + _NBUF - 1
                if j < n_chunks:
                    if i >= 1:
                        write(i - 1).wait()
                    read(j).start()
            for i in range(max(0, n_chunks - _NBUF), n_chunks):
                write(i).wait()

    return sc_copy


def kernel(x, emb_weight):
    seq_len = x.shape[1]
    n_embd = emb_weight.shape[1]
    return _make_sc_copy(seq_len, n_embd, emb_weight.dtype)(emb_weight)


# speedup vs baseline: 1.0599x; 1.0599x over previous
import functools

import jax
import jax.numpy as jnp
from jax import lax
from jax.experimental import pallas as pl
from jax.experimental.pallas import tpu as pltpu
from jax.experimental.pallas import tpu_sc as plsc

_NUM_CORES = 2
_NUM_SUBCORES = 16
_NUM_WORKERS = _NUM_CORES * _NUM_SUBCORES
_CHUNK_ROWS = 16
_NBUF = 3


def _make_sc_copy(seq_len, n_embd, dtype):
    rows_per_w = seq_len // _NUM_WORKERS
    n_chunks = rows_per_w // _CHUNK_ROWS
    mesh = plsc.VectorSubcoreMesh(
        core_axis_name="c", subcore_axis_name="s"
    )

    @functools.partial(
        pl.kernel,
        mesh=mesh,
        out_type=jax.ShapeDtypeStruct((seq_len, n_embd), dtype),
        scratch_types=(
            [pltpu.VMEM_SHARED((_NBUF, _NUM_SUBCORES, _CHUNK_ROWS, n_embd), dtype)]
            + [pltpu.SemaphoreType.DMA] * (2 * _NBUF)
        ),
    )
    def sc_copy(table_hbm, out_hbm, spbuf, *sems):
        rsems = sems[:_NBUF]
        wsems = sems[_NBUF:]
        s = lax.axis_index("s")
        wid = s * _NUM_CORES + lax.axis_index("c")
        base = wid * rows_per_w

        def read(i):
            return pltpu.make_async_copy(
                table_hbm.at[pl.ds(base + i * _CHUNK_ROWS, _CHUNK_ROWS)],
                spbuf.at[i % _NBUF, s],
                rsems[i % _NBUF],
            )

        def write(i):
            return pltpu.make_async_copy(
                spbuf.at[i % _NBUF, s],
                out_hbm.at[pl.ds(base + i * _CHUNK_ROWS, _CHUNK_ROWS)],
                wsems[i % _NBUF],
            )

        def xfer(i, bi, src_is_table):
            if src_is_table:
                return pltpu.make_async_copy(
                    table_hbm.at[pl.ds(base + i * _CHUNK_ROWS, _CHUNK_ROWS)],
                    spbuf.at[bi, s],
                    rsems[bi],
                )
            return pltpu.make_async_copy(
                spbuf.at[bi, s],
                out_hbm.at[pl.ds(base + i * _CHUNK_ROWS, _CHUNK_ROWS)],
                wsems[bi],
            )

        read(0).start()
        read(1).start()
        read(0).wait()
        write(0).start()
        read(2).start()

        n_blocks = (n_chunks - 1) // _NBUF

        def block(k, _):
            for b in range(_NBUF):
                i = 1 + k * _NBUF + b
                bi = (1 + b) % _NBUF
                xfer(i, bi, True).wait()
                xfer(i, bi, False).start()

                @pl.when(i + _NBUF - 1 < n_chunks)
                def _():
                    xfer(i - 1, b % _NBUF, False).wait()
                    xfer(i + _NBUF - 1, b % _NBUF, True).start()
            return _

        lax.fori_loop(0, n_blocks, block, None)

        for i in range(n_chunks - _NBUF, n_chunks):
            write(i).wait()

    return sc_copy


def kernel(x, emb_weight):
    seq_len = x.shape[1]
    n_embd = emb_weight.shape[1]
    return _make_sc_copy(seq_len, n_embd, emb_weight.dtype)(emb_weight)


# speedup vs baseline: 1.0646x; 1.0044x over previous
import functools

import jax
from jax import lax
from jax.experimental import pallas as pl
from jax.experimental.pallas import tpu as pltpu
from jax.experimental.pallas import tpu_sc as plsc

_NUM_CORES = 2
_NUM_SUBCORES = 16
_NUM_WORKERS = _NUM_CORES * _NUM_SUBCORES
_CHUNK_ROWS = 16
_NBUF = 2


def _make_sc_copy(seq_len, n_embd, dtype):
    rows_per_w = seq_len // _NUM_WORKERS
    half_rows = rows_per_w // 2
    n_chunks = half_rows // _CHUNK_ROWS
    mesh = plsc.VectorSubcoreMesh(
        core_axis_name="c", subcore_axis_name="s"
    )

    @functools.partial(
        pl.kernel,
        mesh=mesh,
        out_type=jax.ShapeDtypeStruct((seq_len, n_embd), dtype),
        scratch_types=(
            [pltpu.VMEM((_NBUF, _CHUNK_ROWS, n_embd), dtype)]
            + [pltpu.VMEM_SHARED((_NBUF, _NUM_SUBCORES, _CHUNK_ROWS, n_embd), dtype)]
            + [pltpu.SemaphoreType.DMA] * (4 * _NBUF)
        ),
    )
    def sc_copy(table_hbm, out_hbm, tbuf, spbuf, *sems):
        ra = sems[:_NBUF]
        wa = sems[_NBUF:2 * _NBUF]
        rb = sems[2 * _NBUF:3 * _NBUF]
        wb = sems[3 * _NBUF:]
        s = lax.axis_index("s")
        wid = s * _NUM_CORES + lax.axis_index("c")
        base_a = wid * rows_per_w
        base_b = base_a + half_rows

        def read_a(i):
            return pltpu.make_async_copy(
                table_hbm.at[pl.ds(base_a + i * _CHUNK_ROWS, _CHUNK_ROWS)],
                tbuf.at[i % _NBUF],
                ra[i % _NBUF],
            )

        def write_a(i):
            return pltpu.make_async_copy(
                tbuf.at[i % _NBUF],
                out_hbm.at[pl.ds(base_a + i * _CHUNK_ROWS, _CHUNK_ROWS)],
                wa[i % _NBUF],
            )

        def read_b(i):
            return pltpu.make_async_copy(
                table_hbm.at[pl.ds(base_b + i * _CHUNK_ROWS, _CHUNK_ROWS)],
                spbuf.at[i % _NBUF, s],
                rb[i % _NBUF],
            )

        def write_b(i):
            return pltpu.make_async_copy(
                spbuf.at[i % _NBUF, s],
                out_hbm.at[pl.ds(base_b + i * _CHUNK_ROWS, _CHUNK_ROWS)],
                wb[i % _NBUF],
            )

        read_a(0).start()
        read_b(0).start()
        waited_a = waited_b = 0
        for i in range(n_chunks):
            read_a(i).wait()
            write_a(i).start()
            if i + 1 < n_chunks:
                if i >= 1:
                    write_a(i - 1).wait()
                    waited_a = i
                read_a(i + 1).start()
            read_b(i).wait()
            write_b(i).start()
            if i + 1 < n_chunks:
                if i >= 1:
                    write_b(i - 1).wait()
                    waited_b = i
                read_b(i + 1).start()
        for i in range(waited_a, n_chunks):
            write_a(i).wait()
        for i in range(waited_b, n_chunks):
            write_b(i).wait()

    return sc_copy


def kernel(x, emb_weight):
    seq_len = x.shape[1]
    n_embd = emb_weight.shape[1]
    return _make_sc_copy(seq_len, n_embd, emb_weight.dtype)(emb_weight)
